# baseline (speedup 1.0000x reference)
import jax
import jax.numpy as jnp
from jax.experimental import pallas as pl

N_GRID = 50
USER_ROWS = 2000
ITEM_ROWS = 1000
EMBED = 128


def _copy_kernel(user_in, item_in, user_out, item_out):
    user_out[...] = user_in[...]
    item_out[...] = item_in[...]


def kernel(embed_user, embed_item):
    return tuple(pl.pallas_call(
        _copy_kernel,
        grid=(N_GRID,),
        in_specs=[
            pl.BlockSpec((USER_ROWS, EMBED), lambda i: (i, 0)),
            pl.BlockSpec((ITEM_ROWS, EMBED), lambda i: (i, 0)),
        ],
        out_specs=[
            pl.BlockSpec((USER_ROWS, EMBED), lambda i: (i, 0)),
            pl.BlockSpec((ITEM_ROWS, EMBED), lambda i: (i, 0)),
        ],
        out_shape=[
            jax.ShapeDtypeStruct(embed_user.shape, embed_user.dtype),
            jax.ShapeDtypeStruct(embed_item.shape, embed_item.dtype),
        ],
    )(embed_user, embed_item))
